# baseline (speedup 1.0000x reference)
import functools

import jax
import jax.numpy as jnp
from jax import lax
from jax.experimental import pallas as pl
from jax.experimental.pallas import tpu as pltpu
from jax.experimental.pallas import tpu_sc as plsc

_NC = 2
_NS = 16
_NW = _NC * _NS

_CHUNK = 1280


def _sc_gather(table, idx, embed_dim):
    n = idx.shape[0]
    n_per_w = n // _NW
    n_chunks = n_per_w // _CHUNK

    mesh = plsc.VectorSubcoreMesh(core_axis_name="c", subcore_axis_name="s")

    @functools.partial(
        pl.kernel,
        out_type=jax.ShapeDtypeStruct((n, embed_dim), jnp.float32),
        mesh=mesh,
        scratch_types=[
            pltpu.VMEM((n_per_w,), jnp.int32),
            pltpu.VMEM((_CHUNK, embed_dim), jnp.float32),
            pltpu.VMEM((_CHUNK, embed_dim), jnp.float32),
            pltpu.SemaphoreType.DMA,
            pltpu.SemaphoreType.DMA,
        ],
        compiler_params=pltpu.CompilerParams(use_tc_tiling_on_sc=False),
    )
    def k(table_hbm, idx_hbm, out_hbm, idx_v, rows0, rows1, sem0, sem1):
        wid = lax.axis_index("s") * _NC + lax.axis_index("c")
        base = wid * n_per_w
        rows = (rows0, rows1)
        sems = (sem0, sem1)

        pltpu.sync_copy(idx_hbm.at[pl.ds(base, n_per_w)], idx_v)

        def fire(g, buf, sem):
            pltpu.async_copy(table_hbm.at[idx_v.at[pl.ds(g * _CHUNK, _CHUNK)]], buf, sem)

        fire(0, rows0, sem0)

        def step(g, carry):
            for b in range(2):

                @pl.when(g % 2 == b)
                def _():
                    buf, sem = rows[b], sems[b]
                    pltpu.make_async_copy(out_hbm.at[pl.ds(0, _CHUNK)], buf, sem).wait()

                    @pl.when(g < n_chunks - 1)
                    def _():
                        fire(g + 1, rows[1 - b], sems[1 - b])

                    pltpu.sync_copy(buf, out_hbm.at[pl.ds(base + g * _CHUNK, _CHUNK)])

            return carry

        lax.fori_loop(0, n_chunks, step, 0)

    return k(table, idx)


def kernel(inputs, embedding_matrix):
    batch, seq = inputs.shape
    vocab, embed_dim = embedding_matrix.shape
    n = batch * seq
    out = _sc_gather(embedding_matrix, inputs.astype(jnp.int32).reshape(n), embed_dim)
    return out.reshape(batch, seq, embed_dim)


# speedup vs baseline: 1.0050x; 1.0050x over previous
import functools

import jax
import jax.numpy as jnp
from jax import lax
from jax.experimental import pallas as pl
from jax.experimental.pallas import tpu as pltpu
from jax.experimental.pallas import tpu_sc as plsc

_NC = 2
_NS = 16
_NW = _NC * _NS

_CHUNK = 1280


def _sc_gather(table, idx, embed_dim):
    n = idx.shape[0]
    n_per_w = n // _NW
    n_chunks = n_per_w // _CHUNK

    mesh = plsc.VectorSubcoreMesh(core_axis_name="c", subcore_axis_name="s")

    @functools.partial(
        pl.kernel,
        out_type=jax.ShapeDtypeStruct((n, embed_dim), jnp.float32),
        mesh=mesh,
        scratch_types=[
            pltpu.VMEM((n_per_w,), jnp.int32),
            pltpu.VMEM((_CHUNK, embed_dim), jnp.float32),
            pltpu.VMEM((_CHUNK, embed_dim), jnp.float32),
            pltpu.SemaphoreType.DMA,
            pltpu.SemaphoreType.DMA,
        ],
        compiler_params=pltpu.CompilerParams(use_tc_tiling_on_sc=False),
    )
    def k(table_hbm, idx_hbm, out_hbm, idx_v, rows0, rows1, sem0, sem1):
        wid = lax.axis_index("s") * _NC + lax.axis_index("c")
        base = wid * n_per_w
        rows = (rows0, rows1)
        sems = (sem0, sem1)

        pltpu.sync_copy(idx_hbm.at[pl.ds(base, n_per_w)], idx_v)

        def fire(g, buf, sem):
            pltpu.async_copy(table_hbm.at[idx_v.at[pl.ds(g * _CHUNK, _CHUNK)]], buf, sem)

        fire(0, rows0, sem0)

        def step(g, carry):
            for b in range(2):

                @pl.when(g % 2 == b)
                def _():
                    buf, sem = rows[b], sems[b]

                    @pl.when(g < n_chunks - 1)
                    def _():
                        fire(g + 1, rows[1 - b], sems[1 - b])

                    pltpu.make_async_copy(out_hbm.at[pl.ds(0, _CHUNK)], buf, sem).wait()

                    pltpu.sync_copy(buf, out_hbm.at[pl.ds(base + g * _CHUNK, _CHUNK)])

            return carry

        lax.fori_loop(0, n_chunks, step, 0)

    return k(table, idx)


def kernel(inputs, embedding_matrix):
    batch, seq = inputs.shape
    vocab, embed_dim = embedding_matrix.shape
    n = batch * seq
    out = _sc_gather(embedding_matrix, inputs.astype(jnp.int32).reshape(n), embed_dim)
    return out.reshape(batch, seq, embed_dim)


# speedup vs baseline: 1.0054x; 1.0004x over previous
import functools

import jax
import jax.numpy as jnp
from jax import lax
from jax.experimental import pallas as pl
from jax.experimental.pallas import tpu as pltpu
from jax.experimental.pallas import tpu_sc as plsc

_NC = 2
_NS = 16
_NW = _NC * _NS

_RC = 4


def _sc_gather(table, idx, embed_dim):
    batch, seq = idx.shape
    rows_per_w = batch // _NW
    n_chunks = rows_per_w // _RC

    mesh = plsc.VectorSubcoreMesh(core_axis_name="c", subcore_axis_name="s")

    @functools.partial(
        pl.kernel,
        out_type=jax.ShapeDtypeStruct((batch, seq, embed_dim), jnp.float32),
        mesh=mesh,
        scratch_types=[
            pltpu.VMEM((rows_per_w, seq), jnp.int32),
            pltpu.VMEM((_RC, seq, embed_dim), jnp.float32),
            pltpu.VMEM((_RC, seq, embed_dim), jnp.float32),
            pltpu.SemaphoreType.DMA,
            pltpu.SemaphoreType.DMA,
        ],
        compiler_params=pltpu.CompilerParams(use_tc_tiling_on_sc=False),
    )
    def k(table_hbm, idx_hbm, out_hbm, idx_v, rows0, rows1, sem0, sem1):
        wid = lax.axis_index("s") * _NC + lax.axis_index("c")
        base = wid * rows_per_w
        rows = (rows0, rows1)
        sems = (sem0, sem1)

        pltpu.sync_copy(idx_hbm.at[pl.ds(base, rows_per_w)], idx_v)

        def fire(c, buf, sem):
            for j in range(_RC):
                pltpu.async_copy(
                    table_hbm.at[idx_v.at[c * _RC + j]],
                    buf.at[j],
                    sem,
                )

        fire(0, rows0, sem0)

        def step(c, carry):
            for b in range(2):

                @pl.when(c % 2 == b)
                def _():
                    buf, sem = rows[b], sems[b]

                    @pl.when(c < n_chunks - 1)
                    def _():
                        fire(c + 1, rows[1 - b], sems[1 - b])

                    pltpu.make_async_copy(out_hbm.at[pl.ds(0, _RC)], buf, sem).wait()

                    pltpu.sync_copy(buf, out_hbm.at[pl.ds(base + c * _RC, _RC)])

            return carry

        lax.fori_loop(0, n_chunks, step, 0)

    return k(table, idx)


def kernel(inputs, embedding_matrix):
    vocab, embed_dim = embedding_matrix.shape
    return _sc_gather(embedding_matrix, inputs.astype(jnp.int32), embed_dim)
